# baseline (speedup 1.0000x reference)
import functools

import jax
import jax.numpy as jnp
from jax import lax
from jax.experimental import pallas as pl
from jax.experimental.pallas import tpu as pltpu

F32 = jnp.float32
BE = 1024
BN = 512


def _dot(a, b):
    return jnp.dot(a, b, preferred_element_type=F32)


def _ln(h, lns, lnb):
    mu = jnp.mean(h, axis=-1, keepdims=True)
    var = jnp.mean((h - mu) ** 2, axis=-1, keepdims=True)
    return (h - mu) * lax.rsqrt(var + 1e-5) * lns + lnb


def _feat_stats_body(gs_ref, gr_ref, f_ref, o_ref):
    i = pl.program_id(0)

    @pl.when(i == 0)
    def _():
        o_ref[...] = jnp.zeros_like(o_ref)

    rel = gs_ref[...] - gr_ref[...]
    lane = lax.broadcasted_iota(jnp.int32, rel.shape, 1)
    r2 = rel * rel
    nw = jnp.sqrt(jnp.sum(jnp.where(lane < 3, r2, 0.0), axis=1, keepdims=True))
    nm = jnp.sqrt(jnp.sum(jnp.where((lane >= 3) & (lane < 5), r2, 0.0), axis=1, keepdims=True))
    f = rel + nw * (lane == 5) + nm * (lane == 6)
    f_ref[...] = f
    o_ref[0:1, :] += jnp.sum(f, axis=0, keepdims=True)
    o_ref[1:2, :] += jnp.sum(f * f, axis=0, keepdims=True)


def _stats_body(f_ref, o_ref):
    i = pl.program_id(0)

    @pl.when(i == 0)
    def _():
        o_ref[...] = jnp.zeros_like(o_ref)

    f = f_ref[...]
    o_ref[0:1, :] += jnp.sum(f, axis=0, keepdims=True)
    o_ref[1:2, :] += jnp.sum(f * f, axis=0, keepdims=True)


def _encode_body(f_ref, mean_ref, scale_ref, w1_ref, b1_ref, w2_ref, b2_ref,
                 w3_ref, b3_ref, lns_ref, lnb_ref, o_ref):
    f = (f_ref[...] - mean_ref[...]) * scale_ref[...]
    h = jnp.maximum(_dot(f, w1_ref[...]) + b1_ref[...], 0.0)
    h = jnp.maximum(_dot(h, w2_ref[...]) + b2_ref[...], 0.0)
    h = _dot(h, w3_ref[...]) + b3_ref[...]
    o_ref[...] = _ln(h, lns_ref[...], lnb_ref[...])


def _edge_step_body(gs_ref, gr_ref, e_ref, w1a_ref, w1b_ref, w1c_ref, b1_ref,
                    w2_ref, b2_ref, w3_ref, b3_ref, lns_ref, lnb_ref, o_ref):
    e = e_ref[...]
    h = _dot(gs_ref[...], w1a_ref[...]) + _dot(gr_ref[...], w1b_ref[...])
    h = jnp.maximum(h + _dot(e, w1c_ref[...]) + b1_ref[...], 0.0)
    h = jnp.maximum(_dot(h, w2_ref[...]) + b2_ref[...], 0.0)
    h = _dot(h, w3_ref[...]) + b3_ref[...]
    o_ref[...] = e + _ln(h, lns_ref[...], lnb_ref[...])


def _node_step_body(x_ref, a0_ref, a1_ref, w1a_ref, w1b_ref, b1_ref,
                    w2_ref, b2_ref, w3_ref, b3_ref, lns_ref, lnb_ref, o_ref):
    x = x_ref[...]
    agg = a0_ref[...] + a1_ref[...]
    h = jnp.maximum(_dot(x, w1a_ref[...]) + _dot(agg, w1b_ref[...]) + b1_ref[...], 0.0)
    h = jnp.maximum(_dot(h, w2_ref[...]) + b2_ref[...], 0.0)
    h = _dot(h, w3_ref[...]) + b3_ref[...]
    o_ref[...] = x + _ln(h, lns_ref[...], lnb_ref[...])


def _decoder_body(x_ref, w1_ref, b1_ref, w2_ref, b2_ref, w3_ref, b3_ref, o_ref):
    h = jnp.maximum(_dot(x_ref[...], w1_ref[...]) + b1_ref[...], 0.0)
    h = jnp.maximum(_dot(h, w2_ref[...]) + b2_ref[...], 0.0)
    o_ref[...] = _dot(h, w3_ref[...]) + b3_ref[...]


def _full(shape):
    return pl.BlockSpec(shape, lambda i: (0, 0))


def _rows(bs, w):
    return pl.BlockSpec((bs, w), lambda i: (i, 0))


def _tc_call(body, grid, in_specs, out_specs, out_shape):
    return pl.pallas_call(
        body, grid=grid, in_specs=in_specs, out_specs=out_specs,
        out_shape=out_shape)


def kernel(world_pos, prev_world_pos, mesh_pos, params, node_type, cells,
           is_training=True):
    N = world_pos.shape[0]
    C = cells.shape[0]
    NTS = 9
    E0 = 6 * C
    EPAD = ((E0 + 4095) // 4096) * 4096
    NPAD = ((N + BN - 1) // BN) * BN
    DUMMY = N

    p2 = jnp.concatenate([cells[:, [0, 1]], cells[:, [1, 2]], cells[:, [2, 0]]], axis=0)
    lo = jnp.minimum(p2[:, 0], p2[:, 1])
    hi = jnp.maximum(p2[:, 0], p2[:, 1])
    sk = jnp.sort(lo * N + hi)
    lo_s = sk // N
    hi_s = sk - lo_s * N
    first = jnp.concatenate([jnp.ones((1,), bool), sk[1:] != sk[:-1]])
    valid = first & (lo_s != hi_s)
    lo_v = jnp.where(valid, lo_s, 0)
    hi_v = jnp.where(valid, hi_s, 0)
    pad = EPAD - E0
    zpad = jnp.zeros((pad,), jnp.int32)
    send_g = jnp.concatenate([lo_v, hi_v, zpad])
    recv_g = jnp.concatenate([hi_v, lo_v, zpad])
    dm = jnp.full_like(lo_s, DUMMY)
    recv_s = jnp.concatenate([jnp.where(valid, hi_s, dm), jnp.where(valid, lo_s, dm),
                              jnp.full((pad,), DUMMY, jnp.int32)])
    ecount = 2.0 * jnp.sum(valid).astype(F32)

    def b2d(v):
        return v.reshape(1, -1)

    pn = params["node_encoder"]
    pe = params["edge_encoder"]
    pd = params["decoder"]
    w1n = jnp.pad(pn["W"][0], ((0, 16 - 12), (0, 0)))
    perm = jnp.array([0, 1, 2, 4, 5, 3, 6], jnp.int32)
    w1e = jnp.pad(pe["W"][0][perm], ((0, 16 - 7), (0, 0)))
    w3d = jnp.pad(pd["W"][2], ((0, 0), (0, 128 - 3)))
    b3d = jnp.pad(pd["b"][2], (0, 128 - 3))

    velocity = world_pos - prev_world_pos
    one_hot = jax.nn.one_hot(node_type[:, 0], NTS, dtype=F32)
    nf = jnp.concatenate([velocity, one_hot], axis=-1)
    nf = jnp.pad(nf, ((0, NPAD - N), (0, 16 - 12)))

    pos16 = jnp.zeros((NPAD, 16), F32)
    pos16 = pos16.at[:N, 0:3].set(world_pos).at[:N, 3:5].set(mesh_pos)

    gp_s = jnp.take(pos16, send_g, axis=0)
    gp_r = jnp.take(pos16, recv_g, axis=0)

    ef, est = _tc_call(
        _feat_stats_body, (EPAD // BE,),
        [_rows(BE, 16), _rows(BE, 16)],
        [_rows(BE, 16), pl.BlockSpec((8, 16), lambda i: (0, 0))],
        [jax.ShapeDtypeStruct((EPAD, 16), F32), jax.ShapeDtypeStruct((8, 16), F32)],
    )(gp_s, gp_r)
    emean = est[0] / ecount
    evar = jnp.maximum(est[1] / ecount - emean * emean, 0.0)
    escale = 1.0 / jnp.maximum(jnp.sqrt(evar), 1e-8)

    nst = _tc_call(
        _stats_body, (NPAD // BN,),
        [_rows(BN, 16)],
        pl.BlockSpec((8, 16), lambda i: (0, 0)),
        jax.ShapeDtypeStruct((8, 16), F32),
    )(nf)
    nmean = nst[0] / N
    nvar = jnp.maximum(nst[1] / N - nmean * nmean, 0.0)
    nscale = 1.0 / jnp.maximum(jnp.sqrt(nvar), 1e-8)

    def encode(feats, rows, bs, mean, scale, p, w1):
        ins = [feats, mean.reshape(1, 16), scale.reshape(1, 16), w1, b2d(p["b"][0]),
               p["W"][1], b2d(p["b"][1]), p["W"][2], b2d(p["b"][2]),
               b2d(p["ln_scale"]), b2d(p["ln_bias"])]
        specs = [_rows(bs, 16), _full((1, 16)), _full((1, 16)), _full((16, 128)),
                 _full((1, 128)), _full((128, 128)), _full((1, 128)),
                 _full((128, 128)), _full((1, 128)), _full((1, 128)), _full((1, 128))]
        return _tc_call(_encode_body, (rows // bs,), specs, _rows(bs, 128),
                        jax.ShapeDtypeStruct((rows, 128), F32))(*ins)

    x = encode(nf, NPAD, BN, nmean, nscale, pn, w1n)
    e = encode(ef, EPAD, BE, emean, escale, pe, w1e)

    wfull = _full((128, 128))
    bfull = _full((1, 128))
    for step in params["steps"]:
        se, sn = step["edge"], step["node"]
        gx_s = jnp.take(x, send_g, axis=0)
        gx_r = jnp.take(x, recv_g, axis=0)
        e = _tc_call(
            _edge_step_body, (EPAD // BE,),
            [_rows(BE, 128)] * 3 + [wfull, wfull, wfull, bfull, wfull, bfull,
                                    wfull, bfull, bfull, bfull],
            _rows(BE, 128),
            jax.ShapeDtypeStruct((EPAD, 128), F32),
        )(gx_s, gx_r, e,
          se["W"][0][:128], se["W"][0][128:256], se["W"][0][256:], b2d(se["b"][0]),
          se["W"][1], b2d(se["b"][1]), se["W"][2], b2d(se["b"][2]),
          b2d(se["ln_scale"]), b2d(se["ln_bias"]))
        agg = jnp.zeros((NPAD, 128), F32).at[recv_s].add(e)
        zagg = jnp.zeros((NPAD, 128), F32)
        x = _tc_call(
            _node_step_body, (NPAD // BN,),
            [_rows(BN, 128)] * 3 + [wfull, wfull, bfull, wfull, bfull,
                                    wfull, bfull, bfull, bfull],
            _rows(BN, 128),
            jax.ShapeDtypeStruct((NPAD, 128), F32),
        )(x, agg, zagg,
          sn["W"][0][:128], sn["W"][0][128:], b2d(sn["b"][0]),
          sn["W"][1], b2d(sn["b"][1]), sn["W"][2], b2d(sn["b"][2]),
          b2d(sn["ln_scale"]), b2d(sn["ln_bias"]))

    y = _tc_call(
        _decoder_body, (NPAD // BN,),
        [_rows(BN, 128), wfull, bfull, wfull, bfull, wfull, bfull],
        _rows(BN, 128),
        jax.ShapeDtypeStruct((NPAD, 128), F32),
    )(x, pd["W"][0], b2d(pd["b"][0]), pd["W"][1], b2d(pd["b"][1]), w3d, b2d(b3d))
    return y[:N, :3]


# speedup vs baseline: 1.8022x; 1.8022x over previous
import functools

import jax
import jax.numpy as jnp
from jax import lax
from jax.experimental import pallas as pl
from jax.experimental.pallas import tpu as pltpu
from jax.experimental.pallas import tpu_sc as plsc

F32 = jnp.float32
BE = 1024
BN = 512
NC, NS = 2, 16
NW = NC * NS
CH = 128


def _dot(a, b):
    return jnp.dot(a, b, preferred_element_type=F32)


def _ln(h, lns, lnb):
    mu = jnp.mean(h, axis=-1, keepdims=True)
    var = jnp.mean((h - mu) ** 2, axis=-1, keepdims=True)
    return (h - mu) * lax.rsqrt(var + 1e-5) * lns + lnb


def _feat_stats_body(gs_ref, gr_ref, f_ref, o_ref):
    i = pl.program_id(0)

    @pl.when(i == 0)
    def _():
        o_ref[...] = jnp.zeros_like(o_ref)

    rel = gs_ref[...] - gr_ref[...]
    lane = lax.broadcasted_iota(jnp.int32, rel.shape, 1)
    r2 = rel * rel
    nw = jnp.sqrt(jnp.sum(jnp.where(lane < 3, r2, 0.0), axis=1, keepdims=True))
    nm = jnp.sqrt(jnp.sum(jnp.where((lane >= 3) & (lane < 5), r2, 0.0), axis=1, keepdims=True))
    f = rel + nw * (lane == 5) + nm * (lane == 6)
    f_ref[...] = f
    o_ref[0:1, :] += jnp.sum(f, axis=0, keepdims=True)
    o_ref[1:2, :] += jnp.sum(f * f, axis=0, keepdims=True)


def _stats_body(f_ref, o_ref):
    i = pl.program_id(0)

    @pl.when(i == 0)
    def _():
        o_ref[...] = jnp.zeros_like(o_ref)

    f = f_ref[...]
    o_ref[0:1, :] += jnp.sum(f, axis=0, keepdims=True)
    o_ref[1:2, :] += jnp.sum(f * f, axis=0, keepdims=True)


def _encode_body(f_ref, mean_ref, scale_ref, w1_ref, b1_ref, w2_ref, b2_ref,
                 w3_ref, b3_ref, lns_ref, lnb_ref, o_ref):
    f = (f_ref[...] - mean_ref[...]) * scale_ref[...]
    h = jnp.maximum(_dot(f, w1_ref[...]) + b1_ref[...], 0.0)
    h = jnp.maximum(_dot(h, w2_ref[...]) + b2_ref[...], 0.0)
    h = _dot(h, w3_ref[...]) + b3_ref[...]
    o_ref[...] = _ln(h, lns_ref[...], lnb_ref[...])


def _edge_step_body(gs_ref, gr_ref, e_ref, w1a_ref, w1b_ref, w1c_ref, b1_ref,
                    w2_ref, b2_ref, w3_ref, b3_ref, lns_ref, lnb_ref, o_ref):
    e = e_ref[...]
    h = _dot(gs_ref[...], w1a_ref[...]) + _dot(gr_ref[...], w1b_ref[...])
    h = jnp.maximum(h + _dot(e, w1c_ref[...]) + b1_ref[...], 0.0)
    h = jnp.maximum(_dot(h, w2_ref[...]) + b2_ref[...], 0.0)
    h = _dot(h, w3_ref[...]) + b3_ref[...]
    o_ref[...] = e + _ln(h, lns_ref[...], lnb_ref[...])


def _node_step_body(x_ref, a0_ref, a1_ref, w1a_ref, w1b_ref, b1_ref,
                    w2_ref, b2_ref, w3_ref, b3_ref, lns_ref, lnb_ref, o_ref):
    x = x_ref[...]
    agg = a0_ref[...] + a1_ref[...]
    h = jnp.maximum(_dot(x, w1a_ref[...]) + _dot(agg, w1b_ref[...]) + b1_ref[...], 0.0)
    h = jnp.maximum(_dot(h, w2_ref[...]) + b2_ref[...], 0.0)
    h = _dot(h, w3_ref[...]) + b3_ref[...]
    o_ref[...] = x + _ln(h, lns_ref[...], lnb_ref[...])


def _decoder_body(x_ref, w1_ref, b1_ref, w2_ref, b2_ref, w3_ref, b3_ref, o_ref):
    h = jnp.maximum(_dot(x_ref[...], w1_ref[...]) + b1_ref[...], 0.0)
    h = jnp.maximum(_dot(h, w2_ref[...]) + b2_ref[...], 0.0)
    o_ref[...] = _dot(h, w3_ref[...]) + b3_ref[...]


@functools.lru_cache(maxsize=None)
def _sc_gather2(npad, d, epad):
    nchunk = epad // CH
    per_w = nchunk // NW
    mesh = plsc.VectorSubcoreMesh(core_axis_name="c", subcore_axis_name="s")

    @functools.partial(
        pl.kernel,
        out_type=[jax.ShapeDtypeStruct((epad, d), F32),
                  jax.ShapeDtypeStruct((epad, d), F32)],
        mesh=mesh,
        scratch_types=[pltpu.VMEM((CH,), jnp.int32),
                       pltpu.VMEM((CH, d), F32),
                       pltpu.SemaphoreType.DMA],
    )
    def k(table, sidx, ridx, outs, outr, idx_v, rows_v, sem):
        wid = lax.axis_index("s") * NC + lax.axis_index("c")
        base = wid * per_w

        def one(idx_hbm, out_hbm):
            def body(j, _):
                ch = base + j
                pltpu.sync_copy(idx_hbm.at[ch], idx_v)
                pltpu.async_copy(table.at[idx_v], rows_v, sem).wait()
                pltpu.sync_copy(rows_v, out_hbm.at[pl.ds(ch * CH, CH)])
                return 0

            lax.fori_loop(0, per_w, body, 0)

        one(sidx, outs)
        one(ridx, outr)

    return k


@functools.lru_cache(maxsize=None)
def _sc_scatter(npad, epad):
    nchunk = epad // CH
    per_w = nchunk // NW
    rows_t = npad // NS
    mesh = plsc.VectorSubcoreMesh(core_axis_name="c", subcore_axis_name="s")

    @functools.partial(
        pl.kernel,
        out_type=jax.ShapeDtypeStruct((NC, npad, 128), F32),
        mesh=mesh,
        scratch_types=[pltpu.VMEM((CH,), jnp.int32),
                       pltpu.VMEM((CH, 128), F32),
                       pltpu.VMEM_SHARED((npad, 128), F32),
                       pltpu.SemaphoreType.DMA],
    )
    def k(e_hbm, idx_hbm, out, idx_v, ebuf, acc, sem):
        c = lax.axis_index("c")
        s = lax.axis_index("s")
        wid = s * NC + c
        base = wid * per_w

        def zrow(i, _):
            for t in range(128 // 16):
                ebuf[i, pl.ds(t * 16, 16)] = jnp.zeros((16,), F32)
            return 0

        lax.fori_loop(0, CH, zrow, 0)

        def zcopy(t, _):
            pltpu.sync_copy(ebuf, acc.at[pl.ds(s * rows_t + t * CH, CH)])
            return 0

        lax.fori_loop(0, rows_t // CH, zcopy, 0)
        plsc.subcore_barrier()

        def body(j, _):
            ch = base + j
            pltpu.sync_copy(idx_hbm.at[ch], idx_v)
            pltpu.sync_copy(e_hbm.at[pl.ds(ch * CH, CH)], ebuf)
            pltpu.sync_copy(ebuf, acc.at[idx_v], add=True)
            return 0

        lax.fori_loop(0, per_w, body, 0)
        plsc.subcore_barrier()
        pltpu.sync_copy(acc.at[pl.ds(s * rows_t, rows_t)],
                        out.at[c, pl.ds(s * rows_t, rows_t)])

    return k


def _full(shape):
    return pl.BlockSpec(shape, lambda i: (0, 0))


def _rows(bs, w):
    return pl.BlockSpec((bs, w), lambda i: (i, 0))


def _tc_call(body, grid, in_specs, out_specs, out_shape):
    return pl.pallas_call(
        body, grid=grid, in_specs=in_specs, out_specs=out_specs,
        out_shape=out_shape)


def kernel(world_pos, prev_world_pos, mesh_pos, params, node_type, cells,
           is_training=True):
    N = world_pos.shape[0]
    C = cells.shape[0]
    NTS = 9
    E0 = 6 * C
    EPAD = ((E0 + 4095) // 4096) * 4096
    NPAD = ((N + BN - 1) // BN) * BN
    DUMMY = N

    p2 = jnp.concatenate([cells[:, [0, 1]], cells[:, [1, 2]], cells[:, [2, 0]]], axis=0)
    lo = jnp.minimum(p2[:, 0], p2[:, 1])
    hi = jnp.maximum(p2[:, 0], p2[:, 1])
    sk = jnp.sort(lo * N + hi)
    lo_s = sk // N
    hi_s = sk - lo_s * N
    first = jnp.concatenate([jnp.ones((1,), bool), sk[1:] != sk[:-1]])
    valid = first & (lo_s != hi_s)
    lo_v = jnp.where(valid, lo_s, 0)
    hi_v = jnp.where(valid, hi_s, 0)
    pad = EPAD - E0
    zpad = jnp.zeros((pad,), jnp.int32)
    send_g = jnp.concatenate([lo_v, hi_v, zpad]).reshape(EPAD // CH, CH)
    recv_g = jnp.concatenate([hi_v, lo_v, zpad]).reshape(EPAD // CH, CH)
    dm = jnp.full_like(lo_s, DUMMY)
    recv_s = jnp.concatenate([jnp.where(valid, hi_s, dm), jnp.where(valid, lo_s, dm),
                              jnp.full((pad,), DUMMY, jnp.int32)]).reshape(EPAD // CH, CH)
    ecount = 2.0 * jnp.sum(valid).astype(F32)

    def b2d(v):
        return v.reshape(1, -1)

    pn = params["node_encoder"]
    pe = params["edge_encoder"]
    pd = params["decoder"]
    w1n = jnp.pad(pn["W"][0], ((0, 16 - 12), (0, 0)))
    perm = jnp.array([0, 1, 2, 4, 5, 3, 6], jnp.int32)
    w1e = jnp.pad(pe["W"][0][perm], ((0, 128 - 7), (0, 0)))
    w3d = jnp.pad(pd["W"][2], ((0, 0), (0, 128 - 3)))
    b3d = jnp.pad(pd["b"][2], (0, 128 - 3))

    velocity = world_pos - prev_world_pos
    one_hot = jax.nn.one_hot(node_type[:, 0], NTS, dtype=F32)
    nf = jnp.concatenate([velocity, one_hot], axis=-1)
    nf = jnp.pad(nf, ((0, NPAD - N), (0, 16 - 12)))

    pos128 = jnp.zeros((NPAD, 128), F32)
    pos128 = pos128.at[:N, 0:3].set(world_pos).at[:N, 3:5].set(mesh_pos)

    gather128 = _sc_gather2(NPAD, 128, EPAD)
    gp_s, gp_r = gather128(pos128, send_g, recv_g)

    ef, est = _tc_call(
        _feat_stats_body, (EPAD // BE,),
        [_rows(BE, 128), _rows(BE, 128)],
        [_rows(BE, 128), pl.BlockSpec((8, 128), lambda i: (0, 0))],
        [jax.ShapeDtypeStruct((EPAD, 128), F32), jax.ShapeDtypeStruct((8, 128), F32)],
    )(gp_s, gp_r)
    emean = est[0] / ecount
    evar = jnp.maximum(est[1] / ecount - emean * emean, 0.0)
    escale = 1.0 / jnp.maximum(jnp.sqrt(evar), 1e-8)

    nst = _tc_call(
        _stats_body, (NPAD // BN,),
        [_rows(BN, 16)],
        pl.BlockSpec((8, 16), lambda i: (0, 0)),
        jax.ShapeDtypeStruct((8, 16), F32),
    )(nf)
    nmean = nst[0] / N
    nvar = jnp.maximum(nst[1] / N - nmean * nmean, 0.0)
    nscale = 1.0 / jnp.maximum(jnp.sqrt(nvar), 1e-8)

    def encode(feats, rows, bs, fw, mean, scale, p, w1):
        ins = [feats, mean.reshape(1, fw), scale.reshape(1, fw), w1, b2d(p["b"][0]),
               p["W"][1], b2d(p["b"][1]), p["W"][2], b2d(p["b"][2]),
               b2d(p["ln_scale"]), b2d(p["ln_bias"])]
        specs = [_rows(bs, fw), _full((1, fw)), _full((1, fw)), _full((fw, 128)),
                 _full((1, 128)), _full((128, 128)), _full((1, 128)),
                 _full((128, 128)), _full((1, 128)), _full((1, 128)), _full((1, 128))]
        return _tc_call(_encode_body, (rows // bs,), specs, _rows(bs, 128),
                        jax.ShapeDtypeStruct((rows, 128), F32))(*ins)

    x = encode(nf, NPAD, BN, 16, nmean, nscale, pn, w1n)
    e = encode(ef, EPAD, BE, 128, emean, escale, pe, w1e)

    wfull = _full((128, 128))
    bfull = _full((1, 128))
    scatter_e = _sc_scatter(NPAD, EPAD)
    for step in params["steps"]:
        se, sn = step["edge"], step["node"]
        gx_s, gx_r = gather128(x, send_g, recv_g)
        e = _tc_call(
            _edge_step_body, (EPAD // BE,),
            [_rows(BE, 128)] * 3 + [wfull, wfull, wfull, bfull, wfull, bfull,
                                    wfull, bfull, bfull, bfull],
            _rows(BE, 128),
            jax.ShapeDtypeStruct((EPAD, 128), F32),
        )(gx_s, gx_r, e,
          se["W"][0][:128], se["W"][0][128:256], se["W"][0][256:], b2d(se["b"][0]),
          se["W"][1], b2d(se["b"][1]), se["W"][2], b2d(se["b"][2]),
          b2d(se["ln_scale"]), b2d(se["ln_bias"]))
        aggs = scatter_e(e, recv_s)
        x = _tc_call(
            _node_step_body, (NPAD // BN,),
            [_rows(BN, 128)] * 3 + [wfull, wfull, bfull, wfull, bfull,
                                    wfull, bfull, bfull, bfull],
            _rows(BN, 128),
            jax.ShapeDtypeStruct((NPAD, 128), F32),
        )(x, aggs[0], aggs[1],
          sn["W"][0][:128], sn["W"][0][128:], b2d(sn["b"][0]),
          sn["W"][1], b2d(sn["b"][1]), sn["W"][2], b2d(sn["b"][2]),
          b2d(sn["ln_scale"]), b2d(sn["ln_bias"]))

    y = _tc_call(
        _decoder_body, (NPAD // BN,),
        [_rows(BN, 128), wfull, bfull, wfull, bfull, wfull, bfull],
        _rows(BN, 128),
        jax.ShapeDtypeStruct((NPAD, 128), F32),
    )(x, pd["W"][0], b2d(pd["b"][0]), pd["W"][1], b2d(pd["b"][1]), w3d, b2d(b3d))
    return y[:N, :3]


# speedup vs baseline: 2.0818x; 1.1552x over previous
import functools

import jax
import jax.numpy as jnp
from jax import lax
from jax.experimental import pallas as pl
from jax.experimental.pallas import tpu as pltpu
from jax.experimental.pallas import tpu_sc as plsc

F32 = jnp.float32
BE = 1024
BN = 512
NC, NS = 2, 16
NW = NC * NS
CH = 128


def _dot(a, b):
    return jnp.dot(a, b, preferred_element_type=F32)


def _ln(h, lns, lnb):
    mu = jnp.mean(h, axis=-1, keepdims=True)
    var = jnp.mean((h - mu) ** 2, axis=-1, keepdims=True)
    return (h - mu) * lax.rsqrt(var + 1e-5) * lns + lnb


def _feat_stats_body(gs_ref, gr_ref, f_ref, o_ref):
    i = pl.program_id(0)

    @pl.when(i == 0)
    def _():
        o_ref[...] = jnp.zeros_like(o_ref)

    rel = gs_ref[...] - gr_ref[...]
    lane = lax.broadcasted_iota(jnp.int32, rel.shape, 1)
    r2 = rel * rel
    nw = jnp.sqrt(jnp.sum(jnp.where(lane < 3, r2, 0.0), axis=1, keepdims=True))
    nm = jnp.sqrt(jnp.sum(jnp.where((lane >= 3) & (lane < 5), r2, 0.0), axis=1, keepdims=True))
    f = rel + nw * (lane == 5) + nm * (lane == 6)
    f_ref[...] = f
    o_ref[0:1, :] += jnp.sum(f, axis=0, keepdims=True)
    o_ref[1:2, :] += jnp.sum(f * f, axis=0, keepdims=True)


def _stats_body(f_ref, o_ref):
    i = pl.program_id(0)

    @pl.when(i == 0)
    def _():
        o_ref[...] = jnp.zeros_like(o_ref)

    f = f_ref[...]
    o_ref[0:1, :] += jnp.sum(f, axis=0, keepdims=True)
    o_ref[1:2, :] += jnp.sum(f * f, axis=0, keepdims=True)


def _encode_body(f_ref, mean_ref, scale_ref, w1_ref, b1_ref, w2_ref, b2_ref,
                 w3_ref, b3_ref, lns_ref, lnb_ref, o_ref):
    f = (f_ref[...] - mean_ref[...]) * scale_ref[...]
    h = jnp.maximum(_dot(f, w1_ref[...]) + b1_ref[...], 0.0)
    h = jnp.maximum(_dot(h, w2_ref[...]) + b2_ref[...], 0.0)
    h = _dot(h, w3_ref[...]) + b3_ref[...]
    o_ref[...] = _ln(h, lns_ref[...], lnb_ref[...])


def _edge_step_body(gs_ref, gr_ref, e_ref, w1a_ref, w1b_ref, w1c_ref, b1_ref,
                    w2_ref, b2_ref, w3_ref, b3_ref, lns_ref, lnb_ref, o_ref):
    e = e_ref[...]
    h = _dot(gs_ref[...], w1a_ref[...]) + _dot(gr_ref[...], w1b_ref[...])
    h = jnp.maximum(h + _dot(e, w1c_ref[...]) + b1_ref[...], 0.0)
    h = jnp.maximum(_dot(h, w2_ref[...]) + b2_ref[...], 0.0)
    h = _dot(h, w3_ref[...]) + b3_ref[...]
    o_ref[...] = e + _ln(h, lns_ref[...], lnb_ref[...])


def _node_step_body(x_ref, a0_ref, a1_ref, w1a_ref, w1b_ref, b1_ref,
                    w2_ref, b2_ref, w3_ref, b3_ref, lns_ref, lnb_ref, o_ref):
    x = x_ref[...]
    agg = a0_ref[...] + a1_ref[...]
    h = jnp.maximum(_dot(x, w1a_ref[...]) + _dot(agg, w1b_ref[...]) + b1_ref[...], 0.0)
    h = jnp.maximum(_dot(h, w2_ref[...]) + b2_ref[...], 0.0)
    h = _dot(h, w3_ref[...]) + b3_ref[...]
    o_ref[...] = x + _ln(h, lns_ref[...], lnb_ref[...])


def _decoder_body(x_ref, w1_ref, b1_ref, w2_ref, b2_ref, w3_ref, b3_ref, o_ref):
    h = jnp.maximum(_dot(x_ref[...], w1_ref[...]) + b1_ref[...], 0.0)
    h = jnp.maximum(_dot(h, w2_ref[...]) + b2_ref[...], 0.0)
    o_ref[...] = _dot(h, w3_ref[...]) + b3_ref[...]


GK = 6
GKS = 2


@functools.lru_cache(maxsize=None)
def _sc_gather2(npad, d, epad):
    nchunk = epad // CH
    per_w = nchunk // NW
    assert per_w % GK == 0
    mesh = plsc.VectorSubcoreMesh(core_axis_name="c", subcore_axis_name="s")

    @functools.partial(
        pl.kernel,
        out_type=[jax.ShapeDtypeStruct((epad, d), F32),
                  jax.ShapeDtypeStruct((epad, d), F32)],
        mesh=mesh,
        scratch_types=[pltpu.VMEM((per_w, CH), jnp.int32),
                       pltpu.VMEM((per_w, CH), jnp.int32),
                       pltpu.VMEM((GK, CH, d), F32),
                       pltpu.SemaphoreType.DMA,
                       pltpu.SemaphoreType.DMA],
    )
    def k(table, sidx, ridx, outs, outr, idxs_v, idxr_v, bufs, gsem, wsem):
        wid = lax.axis_index("s") * NC + lax.axis_index("c")
        base = wid * per_w
        pltpu.sync_copy(sidx.at[wid], idxs_v)
        pltpu.sync_copy(ridx.at[wid], idxr_v)

        def one(idx_v, out_hbm):
            def group(i, _):
                j0 = i * GK
                gds = [pltpu.async_copy(table.at[idx_v.at[j0 + b]],
                                        bufs.at[b], gsem) for b in range(GK)]
                wds = []
                for b in range(GK):
                    gds[b].wait()
                    wds.append(pltpu.async_copy(
                        bufs.at[b], out_hbm.at[pl.ds((base + j0 + b) * CH, CH)],
                        wsem))
                for b in range(GK):
                    wds[b].wait()
                return 0

            lax.fori_loop(0, per_w // GK, group, 0)

        one(idxs_v, outs)
        one(idxr_v, outr)

    return k


@functools.lru_cache(maxsize=None)
def _sc_scatter(npad, epad):
    nchunk = epad // CH
    per_w = nchunk // NW
    assert per_w % GKS == 0
    rows_t = npad // NS
    mesh = plsc.VectorSubcoreMesh(core_axis_name="c", subcore_axis_name="s")

    @functools.partial(
        pl.kernel,
        out_type=jax.ShapeDtypeStruct((NC, npad, 128), F32),
        mesh=mesh,
        scratch_types=[pltpu.VMEM((per_w, CH), jnp.int32),
                       pltpu.VMEM((GKS, CH, 128), F32),
                       pltpu.VMEM_SHARED((npad, 128), F32),
                       pltpu.SemaphoreType.DMA,
                       pltpu.SemaphoreType.DMA],
    )
    def k(e_hbm, idx_hbm, out, idx_all, bufs, acc, gsem, wsem):
        c = lax.axis_index("c")
        s = lax.axis_index("s")
        wid = s * NC + c
        base = wid * per_w
        pltpu.sync_copy(idx_hbm.at[wid], idx_all)

        def zrow(i, _):
            for t in range(128 // 16):
                bufs[0, i, pl.ds(t * 16, 16)] = jnp.zeros((16,), F32)
            return 0

        lax.fori_loop(0, CH, zrow, 0)
        zds = [pltpu.async_copy(bufs.at[0],
                                acc.at[pl.ds(s * rows_t + t * CH, CH)], wsem)
               for t in range(rows_t // CH)]
        for zd in zds:
            zd.wait()
        plsc.subcore_barrier()

        def group(i, _):
            j0 = i * GKS
            lds = [pltpu.async_copy(e_hbm.at[pl.ds((base + j0 + b) * CH, CH)],
                                    bufs.at[b], gsem) for b in range(GKS)]
            sds = []
            for b in range(GKS):
                lds[b].wait()
                sds.append(pltpu.async_copy(bufs.at[b], acc.at[idx_all.at[j0 + b]],
                                            wsem, add=True))
            for b in range(GKS):
                sds[b].wait()
            return 0

        lax.fori_loop(0, per_w // GKS, group, 0)
        plsc.subcore_barrier()
        pltpu.sync_copy(acc.at[pl.ds(s * rows_t, rows_t)],
                        out.at[c, pl.ds(s * rows_t, rows_t)])

    return k


def _full(shape):
    return pl.BlockSpec(shape, lambda i: (0, 0))


def _rows(bs, w):
    return pl.BlockSpec((bs, w), lambda i: (i, 0))


def _tc_call(body, grid, in_specs, out_specs, out_shape):
    return pl.pallas_call(
        body, grid=grid, in_specs=in_specs, out_specs=out_specs,
        out_shape=out_shape)


def kernel(world_pos, prev_world_pos, mesh_pos, params, node_type, cells,
           is_training=True):
    N = world_pos.shape[0]
    C = cells.shape[0]
    NTS = 9
    E0 = 6 * C
    EPAD = ((E0 + 4095) // 4096) * 4096
    NPAD = ((N + BN - 1) // BN) * BN
    DUMMY = N

    p2 = jnp.concatenate([cells[:, [0, 1]], cells[:, [1, 2]], cells[:, [2, 0]]], axis=0)
    lo = jnp.minimum(p2[:, 0], p2[:, 1])
    hi = jnp.maximum(p2[:, 0], p2[:, 1])
    sk = jnp.sort(lo * N + hi)
    lo_s = sk // N
    hi_s = sk - lo_s * N
    first = jnp.concatenate([jnp.ones((1,), bool), sk[1:] != sk[:-1]])
    valid = first & (lo_s != hi_s)
    lo_v = jnp.where(valid, lo_s, 0)
    hi_v = jnp.where(valid, hi_s, 0)
    pad = EPAD - E0
    zpad = jnp.zeros((pad,), jnp.int32)
    send_g = jnp.concatenate([lo_v, hi_v, zpad]).reshape(NW, EPAD // CH // NW, CH)
    recv_g = jnp.concatenate([hi_v, lo_v, zpad]).reshape(NW, EPAD // CH // NW, CH)
    dm = jnp.full_like(lo_s, DUMMY)
    recv_s = jnp.concatenate([jnp.where(valid, hi_s, dm), jnp.where(valid, lo_s, dm),
                              jnp.full((pad,), DUMMY, jnp.int32)]).reshape(
                                  NW, EPAD // CH // NW, CH)
    ecount = 2.0 * jnp.sum(valid).astype(F32)

    def b2d(v):
        return v.reshape(1, -1)

    pn = params["node_encoder"]
    pe = params["edge_encoder"]
    pd = params["decoder"]
    w1n = jnp.pad(pn["W"][0], ((0, 16 - 12), (0, 0)))
    perm = jnp.array([0, 1, 2, 4, 5, 3, 6], jnp.int32)
    w1e = jnp.pad(pe["W"][0][perm], ((0, 128 - 7), (0, 0)))
    w3d = jnp.pad(pd["W"][2], ((0, 0), (0, 128 - 3)))
    b3d = jnp.pad(pd["b"][2], (0, 128 - 3))

    velocity = world_pos - prev_world_pos
    one_hot = jax.nn.one_hot(node_type[:, 0], NTS, dtype=F32)
    nf = jnp.concatenate([velocity, one_hot], axis=-1)
    nf = jnp.pad(nf, ((0, NPAD - N), (0, 16 - 12)))

    pos128 = jnp.zeros((NPAD, 128), F32)
    pos128 = pos128.at[:N, 0:3].set(world_pos).at[:N, 3:5].set(mesh_pos)

    gather128 = _sc_gather2(NPAD, 128, EPAD)
    gp_s, gp_r = gather128(pos128, send_g, recv_g)

    ef, est = _tc_call(
        _feat_stats_body, (EPAD // BE,),
        [_rows(BE, 128), _rows(BE, 128)],
        [_rows(BE, 128), pl.BlockSpec((8, 128), lambda i: (0, 0))],
        [jax.ShapeDtypeStruct((EPAD, 128), F32), jax.ShapeDtypeStruct((8, 128), F32)],
    )(gp_s, gp_r)
    emean = est[0] / ecount
    evar = jnp.maximum(est[1] / ecount - emean * emean, 0.0)
    escale = 1.0 / jnp.maximum(jnp.sqrt(evar), 1e-8)

    nst = _tc_call(
        _stats_body, (NPAD // BN,),
        [_rows(BN, 16)],
        pl.BlockSpec((8, 16), lambda i: (0, 0)),
        jax.ShapeDtypeStruct((8, 16), F32),
    )(nf)
    nmean = nst[0] / N
    nvar = jnp.maximum(nst[1] / N - nmean * nmean, 0.0)
    nscale = 1.0 / jnp.maximum(jnp.sqrt(nvar), 1e-8)

    def encode(feats, rows, bs, fw, mean, scale, p, w1):
        ins = [feats, mean.reshape(1, fw), scale.reshape(1, fw), w1, b2d(p["b"][0]),
               p["W"][1], b2d(p["b"][1]), p["W"][2], b2d(p["b"][2]),
               b2d(p["ln_scale"]), b2d(p["ln_bias"])]
        specs = [_rows(bs, fw), _full((1, fw)), _full((1, fw)), _full((fw, 128)),
                 _full((1, 128)), _full((128, 128)), _full((1, 128)),
                 _full((128, 128)), _full((1, 128)), _full((1, 128)), _full((1, 128))]
        return _tc_call(_encode_body, (rows // bs,), specs, _rows(bs, 128),
                        jax.ShapeDtypeStruct((rows, 128), F32))(*ins)

    x = encode(nf, NPAD, BN, 16, nmean, nscale, pn, w1n)
    e = encode(ef, EPAD, BE, 128, emean, escale, pe, w1e)

    wfull = _full((128, 128))
    bfull = _full((1, 128))
    scatter_e = _sc_scatter(NPAD, EPAD)
    for step in params["steps"]:
        se, sn = step["edge"], step["node"]
        gx_s, gx_r = gather128(x, send_g, recv_g)
        e = _tc_call(
            _edge_step_body, (EPAD // BE,),
            [_rows(BE, 128)] * 3 + [wfull, wfull, wfull, bfull, wfull, bfull,
                                    wfull, bfull, bfull, bfull],
            _rows(BE, 128),
            jax.ShapeDtypeStruct((EPAD, 128), F32),
        )(gx_s, gx_r, e,
          se["W"][0][:128], se["W"][0][128:256], se["W"][0][256:], b2d(se["b"][0]),
          se["W"][1], b2d(se["b"][1]), se["W"][2], b2d(se["b"][2]),
          b2d(se["ln_scale"]), b2d(se["ln_bias"]))
        aggs = scatter_e(e, recv_s)
        x = _tc_call(
            _node_step_body, (NPAD // BN,),
            [_rows(BN, 128)] * 3 + [wfull, wfull, bfull, wfull, bfull,
                                    wfull, bfull, bfull, bfull],
            _rows(BN, 128),
            jax.ShapeDtypeStruct((NPAD, 128), F32),
        )(x, aggs[0], aggs[1],
          sn["W"][0][:128], sn["W"][0][128:], b2d(sn["b"][0]),
          sn["W"][1], b2d(sn["b"][1]), sn["W"][2], b2d(sn["b"][2]),
          b2d(sn["ln_scale"]), b2d(sn["ln_bias"]))

    y = _tc_call(
        _decoder_body, (NPAD // BN,),
        [_rows(BN, 128), wfull, bfull, wfull, bfull, wfull, bfull],
        _rows(BN, 128),
        jax.ShapeDtypeStruct((NPAD, 128), F32),
    )(x, pd["W"][0], b2d(pd["b"][0]), pd["W"][1], b2d(pd["b"][1]), w3d, b2d(b3d))
    return y[:N, :3]


# speedup vs baseline: 2.2550x; 1.0832x over previous
import functools

import jax
import jax.numpy as jnp
from jax import lax
from jax.experimental import pallas as pl
from jax.experimental.pallas import tpu as pltpu
from jax.experimental.pallas import tpu_sc as plsc

F32 = jnp.float32
BE = 1024
BN = 512
NC, NS = 2, 16
NW = NC * NS
CH = 128


def _dot(a, b):
    return jnp.dot(a, b, preferred_element_type=F32)


def _ln(h, lns, lnb):
    mu = jnp.mean(h, axis=-1, keepdims=True)
    var = jnp.mean((h - mu) ** 2, axis=-1, keepdims=True)
    return (h - mu) * lax.rsqrt(var + 1e-5) * lns + lnb


def _feat_stats_body(gs_ref, gr_ref, f_ref, o_ref):
    i = pl.program_id(0)

    @pl.when(i == 0)
    def _():
        o_ref[...] = jnp.zeros_like(o_ref)

    rel = gs_ref[...] - gr_ref[...]
    lane = lax.broadcasted_iota(jnp.int32, rel.shape, 1)
    r2 = rel * rel
    nw = jnp.sqrt(jnp.sum(jnp.where(lane < 3, r2, 0.0), axis=1, keepdims=True))
    nm = jnp.sqrt(jnp.sum(jnp.where((lane >= 4) & (lane < 6), r2, 0.0), axis=1, keepdims=True))
    f = rel + nw * (lane == 3) + nm * (lane == 6)
    f_ref[...] = f
    o_ref[0:1, :] += jnp.sum(f, axis=0, keepdims=True)
    o_ref[1:2, :] += jnp.sum(f * f, axis=0, keepdims=True)


def _stats_body(f_ref, o_ref):
    i = pl.program_id(0)

    @pl.when(i == 0)
    def _():
        o_ref[...] = jnp.zeros_like(o_ref)

    f = f_ref[...]
    o_ref[0:1, :] += jnp.sum(f, axis=0, keepdims=True)
    o_ref[1:2, :] += jnp.sum(f * f, axis=0, keepdims=True)


def _encode_body(f_ref, mean_ref, scale_ref, w1_ref, b1_ref, w2_ref, b2_ref,
                 w3_ref, b3_ref, lns_ref, lnb_ref, o_ref):
    f = (f_ref[...] - mean_ref[...]) * scale_ref[...]
    h = jnp.maximum(_dot(f, w1_ref[...]) + b1_ref[...], 0.0)
    h = jnp.maximum(_dot(h, w2_ref[...]) + b2_ref[...], 0.0)
    h = _dot(h, w3_ref[...]) + b3_ref[...]
    o_ref[...] = _ln(h, lns_ref[...], lnb_ref[...])


def _edge_step_body(gs_ref, gr_ref, e_ref, w1ab_ref, w1c_ref, b1_ref,
                    w2_ref, b2_ref, w3_ref, b3_ref, lns_ref, lnb_ref, o_ref):
    e = e_ref[...]
    h = _dot(jnp.concatenate([gs_ref[...], gr_ref[...]], axis=1), w1ab_ref[...])
    h = jnp.maximum((h + _dot(e, w1c_ref[...])) + b1_ref[...], 0.0)
    h = jnp.maximum(_dot(h, w2_ref[...]) + b2_ref[...], 0.0)
    h = _dot(h, w3_ref[...]) + b3_ref[...]
    o_ref[...] = e + _ln(h, lns_ref[...], lnb_ref[...])


def _node_step_body(x_ref, a0_ref, a1_ref, w1_ref, b1_ref,
                    w2_ref, b2_ref, w3_ref, b3_ref, lns_ref, lnb_ref, o_ref):
    x = x_ref[...]
    agg = a0_ref[...] + a1_ref[...]
    h = jnp.maximum(_dot(jnp.concatenate([x, agg], axis=1), w1_ref[...]) + b1_ref[...], 0.0)
    h = jnp.maximum(_dot(h, w2_ref[...]) + b2_ref[...], 0.0)
    h = _dot(h, w3_ref[...]) + b3_ref[...]
    o_ref[...] = x + _ln(h, lns_ref[...], lnb_ref[...])


def _decoder_body(x_ref, w1_ref, b1_ref, w2_ref, b2_ref, w3_ref, b3_ref, o_ref):
    h = jnp.maximum(_dot(x_ref[...], w1_ref[...]) + b1_ref[...], 0.0)
    h = jnp.maximum(_dot(h, w2_ref[...]) + b2_ref[...], 0.0)
    o_ref[...] = _dot(h, w3_ref[...]) + b3_ref[...]


GK = 3


@functools.lru_cache(maxsize=None)
def _sc_gather2(npad, d, epad, dt):
    nchunk = epad // CH
    per_w = nchunk // NW
    ngrp = per_w // GK
    assert per_w % GK == 0 and ngrp % 2 == 0 and ngrp >= 4
    nbody = ngrp // 2 - 2
    mesh = plsc.VectorSubcoreMesh(core_axis_name="c", subcore_axis_name="s")

    @functools.partial(
        pl.kernel,
        out_type=[jax.ShapeDtypeStruct((epad, d), dt),
                  jax.ShapeDtypeStruct((epad, d), dt)],
        mesh=mesh,
        scratch_types=[pltpu.VMEM((per_w, CH), jnp.int32),
                       pltpu.VMEM((per_w, CH), jnp.int32),
                       pltpu.VMEM((2 * GK, CH, d), dt),
                       pltpu.SemaphoreType.DMA,
                       pltpu.SemaphoreType.DMA,
                       pltpu.SemaphoreType.DMA,
                       pltpu.SemaphoreType.DMA],
    )
    def k(table, sidx, ridx, outs, outr, idxs_v, idxr_v, bufs,
          gsa, gsb, wsa, wsb):
        wid = lax.axis_index("s") * NC + lax.axis_index("c")
        base = wid * per_w
        pltpu.sync_copy(sidx.at[wid], idxs_v)
        pltpu.sync_copy(ridx.at[wid], idxr_v)
        gsem = [gsa, gsb]
        wsem = [wsa, wsb]

        def one(idx_v, out_hbm):
            def fire_g(st, g):
                for b in range(GK):
                    pltpu.async_copy(table.at[idx_v.at[g * GK + b]],
                                     bufs.at[st * GK + b], gsem[st])

            def fire_w(st, g):
                for b in range(GK):
                    pltpu.async_copy(
                        bufs.at[st * GK + b],
                        out_hbm.at[pl.ds((base + g * GK + b) * CH, CH)],
                        wsem[st])

            def drain(sem):
                for b in range(GK):
                    pltpu.make_async_copy(table.at[pl.ds(0, CH)],
                                          bufs.at[b], sem).wait()

            fire_g(0, 0)
            fire_g(1, 1)
            drain(gsem[0])
            fire_w(0, 0)
            drain(wsem[0])
            fire_g(0, 2)

            def body(gg, _):
                drain(gsem[1])
                fire_w(1, 2 * gg + 1)
                drain(wsem[1])
                fire_g(1, 2 * gg + 3)
                drain(gsem[0])
                fire_w(0, 2 * gg + 2)
                drain(wsem[0])
                fire_g(0, 2 * gg + 4)
                return 0

            lax.fori_loop(0, nbody, body, 0)
            drain(gsem[1])
            fire_w(1, ngrp - 3)
            drain(wsem[1])
            fire_g(1, ngrp - 1)
            drain(gsem[0])
            fire_w(0, ngrp - 2)
            drain(wsem[0])
            drain(gsem[1])
            fire_w(1, ngrp - 1)
            drain(wsem[1])

        one(idxs_v, outs)
        one(idxr_v, outr)

    return k


@functools.lru_cache(maxsize=None)
def _sc_scatter(npad, epad):
    nchunk = epad // CH
    per_w = nchunk // NW
    assert per_w % 2 == 0 and per_w >= 4
    nbody = per_w // 2 - 2
    rows_t = npad // NS
    mesh = plsc.VectorSubcoreMesh(core_axis_name="c", subcore_axis_name="s")

    @functools.partial(
        pl.kernel,
        out_type=jax.ShapeDtypeStruct((NC, npad, 128), F32),
        mesh=mesh,
        scratch_types=[pltpu.VMEM((per_w, CH), jnp.int32),
                       pltpu.VMEM((2, CH, 128), F32),
                       pltpu.VMEM_SHARED((npad, 128), F32),
                       pltpu.SemaphoreType.DMA,
                       pltpu.SemaphoreType.DMA,
                       pltpu.SemaphoreType.DMA,
                       pltpu.SemaphoreType.DMA],
    )
    def k(e_hbm, idx_hbm, out, idx_all, bufs, acc, lsa, lsb, ssa, ssb):
        c = lax.axis_index("c")
        s = lax.axis_index("s")
        wid = s * NC + c
        base = wid * per_w
        pltpu.sync_copy(idx_hbm.at[wid], idx_all)
        lsem = [lsa, lsb]
        ssem = [ssa, ssb]

        def zrow(i, _):
            for t in range(128 // 16):
                bufs[0, i, pl.ds(t * 16, 16)] = jnp.zeros((16,), F32)
            return 0

        lax.fori_loop(0, CH, zrow, 0)
        zds = [pltpu.async_copy(bufs.at[0],
                                acc.at[pl.ds(s * rows_t + t * CH, CH)], ssa)
               for t in range(rows_t // CH)]
        for zd in zds:
            zd.wait()
        plsc.subcore_barrier()

        def fire_l(st, ch):
            pltpu.async_copy(e_hbm.at[pl.ds((base + ch) * CH, CH)],
                             bufs.at[st], lsem[st])

        def fire_s(st, ch):
            pltpu.async_copy(bufs.at[st], acc.at[idx_all.at[ch]],
                             ssem[st], add=True)

        def drain(sem):
            pltpu.make_async_copy(e_hbm.at[pl.ds(0, CH)],
                                  bufs.at[0], sem).wait()

        fire_l(0, 0)
        fire_l(1, 1)
        drain(lsem[0])
        fire_s(0, 0)
        drain(ssem[0])
        fire_l(0, 2)

        def body(gg, _):
            drain(lsem[1])
            fire_s(1, 2 * gg + 1)
            drain(ssem[1])
            fire_l(1, 2 * gg + 3)
            drain(lsem[0])
            fire_s(0, 2 * gg + 2)
            drain(ssem[0])
            fire_l(0, 2 * gg + 4)
            return 0

        lax.fori_loop(0, nbody, body, 0)
        drain(lsem[1])
        fire_s(1, per_w - 3)
        drain(ssem[1])
        fire_l(1, per_w - 1)
        drain(lsem[0])
        fire_s(0, per_w - 2)
        drain(ssem[0])
        drain(lsem[1])
        fire_s(1, per_w - 1)
        drain(ssem[1])

        plsc.subcore_barrier()
        pltpu.sync_copy(acc.at[pl.ds(s * rows_t, rows_t)],
                        out.at[c, pl.ds(s * rows_t, rows_t)])

    return k


def _full(shape):
    return pl.BlockSpec(shape, lambda i: (0, 0))


def _rows(bs, w):
    return pl.BlockSpec((bs, w), lambda i: (i, 0))


def _tc_call(body, grid, in_specs, out_specs, out_shape):
    return pl.pallas_call(
        body, grid=grid, in_specs=in_specs, out_specs=out_specs,
        out_shape=out_shape)


def kernel(world_pos, prev_world_pos, mesh_pos, params, node_type, cells,
           is_training=True):
    N = world_pos.shape[0]
    C = cells.shape[0]
    NTS = 9
    E0 = 6 * C
    EPAD = ((E0 + 4095) // 4096) * 4096
    NPAD = ((N + BN - 1) // BN) * BN
    DUMMY = N

    p2 = jnp.concatenate([cells[:, [0, 1]], cells[:, [1, 2]], cells[:, [2, 0]]], axis=0)
    lo = jnp.minimum(p2[:, 0], p2[:, 1])
    hi = jnp.maximum(p2[:, 0], p2[:, 1])
    sk = jnp.sort(lo * N + hi)
    lo_s = sk // N
    hi_s = sk - lo_s * N
    first = jnp.concatenate([jnp.ones((1,), bool), sk[1:] != sk[:-1]])
    valid = first & (lo_s != hi_s)
    lo_v = jnp.where(valid, lo_s, 0)
    hi_v = jnp.where(valid, hi_s, 0)
    pad = EPAD - E0
    zpad = jnp.zeros((pad,), jnp.int32)
    send_g = jnp.concatenate([lo_v, hi_v, zpad]).reshape(NW, EPAD // CH // NW, CH)
    recv_g = jnp.concatenate([hi_v, lo_v, zpad]).reshape(NW, EPAD // CH // NW, CH)
    dm = jnp.full_like(lo_s, DUMMY)
    recv_s = jnp.concatenate([jnp.where(valid, hi_s, dm), jnp.where(valid, lo_s, dm),
                              jnp.full((pad,), DUMMY, jnp.int32)]).reshape(
                                  NW, EPAD // CH // NW, CH)
    ecount = 2.0 * jnp.sum(valid).astype(F32)

    def b2d(v):
        return v.reshape(1, -1)

    pn = params["node_encoder"]
    pe = params["edge_encoder"]
    pd = params["decoder"]
    w1n = jnp.pad(pn["W"][0], ((0, 16 - 12), (0, 0)))
    w1e = jnp.pad(pe["W"][0], ((0, 128 - 7), (0, 0)))
    w3d = jnp.pad(pd["W"][2], ((0, 0), (0, 128 - 3)))
    b3d = jnp.pad(pd["b"][2], (0, 128 - 3))

    velocity = world_pos - prev_world_pos
    one_hot = jax.nn.one_hot(node_type[:, 0], NTS, dtype=F32)
    nf = jnp.concatenate([velocity, one_hot], axis=-1)
    nf = jnp.pad(nf, ((0, NPAD - N), (0, 16 - 12)))

    pos128 = jnp.zeros((NPAD, 128), F32)
    pos128 = pos128.at[:N, 0:3].set(world_pos).at[:N, 4:6].set(mesh_pos)

    gather128 = _sc_gather2(NPAD, 128, EPAD, F32)
    gp_s, gp_r = gather128(pos128, send_g, recv_g)

    ef, est = _tc_call(
        _feat_stats_body, (EPAD // BE,),
        [_rows(BE, 128), _rows(BE, 128)],
        [_rows(BE, 128), pl.BlockSpec((8, 128), lambda i: (0, 0))],
        [jax.ShapeDtypeStruct((EPAD, 128), F32), jax.ShapeDtypeStruct((8, 128), F32)],
    )(gp_s, gp_r)
    emean = est[0] / ecount
    evar = jnp.maximum(est[1] / ecount - emean * emean, 0.0)
    escale = 1.0 / jnp.maximum(jnp.sqrt(evar), 1e-8)

    nst = _tc_call(
        _stats_body, (NPAD // BN,),
        [_rows(BN, 16)],
        pl.BlockSpec((8, 16), lambda i: (0, 0)),
        jax.ShapeDtypeStruct((8, 16), F32),
    )(nf)
    nmean = nst[0] / N
    nvar = jnp.maximum(nst[1] / N - nmean * nmean, 0.0)
    nscale = 1.0 / jnp.maximum(jnp.sqrt(nvar), 1e-8)

    def encode(feats, rows, bs, fw, mean, scale, p, w1):
        ins = [feats, mean.reshape(1, fw), scale.reshape(1, fw), w1, b2d(p["b"][0]),
               p["W"][1], b2d(p["b"][1]), p["W"][2], b2d(p["b"][2]),
               b2d(p["ln_scale"]), b2d(p["ln_bias"])]
        specs = [_rows(bs, fw), _full((1, fw)), _full((1, fw)), _full((fw, 128)),
                 _full((1, 128)), _full((128, 128)), _full((1, 128)),
                 _full((128, 128)), _full((1, 128)), _full((1, 128)), _full((1, 128))]
        return _tc_call(_encode_body, (rows // bs,), specs, _rows(bs, 128),
                        jax.ShapeDtypeStruct((rows, 128), F32))(*ins)

    x = encode(nf, NPAD, BN, 16, nmean, nscale, pn, w1n)
    e = encode(ef, EPAD, BE, 128, emean, escale, pe, w1e)

    wfull = _full((128, 128))
    bfull = _full((1, 128))
    scatter_e = _sc_scatter(NPAD, EPAD)
    for step in params["steps"]:
        se, sn = step["edge"], step["node"]
        gx_s, gx_r = gather128(x, send_g, recv_g)
        e = _tc_call(
            _edge_step_body, (EPAD // BE,),
            [_rows(BE, 128)] * 3 + [_full((256, 128)), wfull, bfull, wfull, bfull,
                                    wfull, bfull, bfull, bfull],
            _rows(BE, 128),
            jax.ShapeDtypeStruct((EPAD, 128), F32),
        )(gx_s, gx_r, e,
          se["W"][0][:256], se["W"][0][256:], b2d(se["b"][0]),
          se["W"][1], b2d(se["b"][1]), se["W"][2], b2d(se["b"][2]),
          b2d(se["ln_scale"]), b2d(se["ln_bias"]))
        aggs = scatter_e(e, recv_s)
        x = _tc_call(
            _node_step_body, (NPAD // BN,),
            [_rows(BN, 128)] * 3 + [_full((256, 128)), bfull, wfull, bfull,
                                    wfull, bfull, bfull, bfull],
            _rows(BN, 128),
            jax.ShapeDtypeStruct((NPAD, 128), F32),
        )(x, aggs[0], aggs[1],
          sn["W"][0], b2d(sn["b"][0]),
          sn["W"][1], b2d(sn["b"][1]), sn["W"][2], b2d(sn["b"][2]),
          b2d(sn["ln_scale"]), b2d(sn["ln_bias"]))

    y = _tc_call(
        _decoder_body, (NPAD // BN,),
        [_rows(BN, 128), wfull, bfull, wfull, bfull, wfull, bfull],
        _rows(BN, 128),
        jax.ShapeDtypeStruct((NPAD, 128), F32),
    )(x, pd["W"][0], b2d(pd["b"][0]), pd["W"][1], b2d(pd["b"][1]), w3d, b2d(b3d))
    return y[:N, :3]
